# Initial kernel scaffold; baseline (speedup 1.0000x reference)
import jax
import jax.numpy as jnp
from jax.experimental import pallas as pl


def kernel(h, edge_index):
    raise NotImplementedError("write your pallas kernel here")


# speedup vs baseline: 1.1042x; 1.1042x over previous
import functools

import jax
import jax.numpy as jnp
from jax import lax
from jax.experimental import pallas as pl
from jax.experimental.pallas import tpu as pltpu
from jax.experimental.pallas import tpu_sc as plsc

N_NODES = 10000
N_EDGES = 320000
D_FEAT = 128

NUM_CORES = 2
NUM_SUBCORES = 16
LANES = 16
NUM_WORKERS = NUM_CORES * NUM_SUBCORES

E_PER_W = N_EDGES // NUM_WORKERS
CHUNK = 80
N_CHUNKS = E_PER_W // CHUNK
GROUPS = CHUNK // LANES


def _dot_scores(urows, vrows, scores):
    for g in range(GROUPS):
        eidx = g * LANES + lax.broadcasted_iota(jnp.int32, (LANES,), 0)

        def body(_, carry):
            acc, dvec = carry
            du = plsc.load_gather(urows, [eidx, dvec])
            dv = plsc.load_gather(vrows, [eidx, dvec])
            return acc + du * dv, dvec + 1

        acc, _ = lax.fori_loop(
            0, D_FEAT, body,
            (jnp.zeros((LANES,), jnp.float32), jnp.zeros((LANES,), jnp.int32)),
        )
        scores[pl.ds(g * LANES, LANES)] = acc


def _sc_body(h_hbm, u_hbm, v_hbm, out_hbm,
             uidx, vidx, urows, vrows, scores, sem_u, sem_v):
    wid = lax.axis_index("s") * NUM_CORES + lax.axis_index("c")
    base = wid * E_PER_W

    def chunk_body(i, carry):
        off = base + i * CHUNK
        pltpu.sync_copy(u_hbm.at[pl.ds(off, CHUNK)], uidx)
        pltpu.sync_copy(v_hbm.at[pl.ds(off, CHUNK)], vidx)
        cu = pltpu.async_copy(h_hbm.at[uidx], urows, sem_u)
        cv = pltpu.async_copy(h_hbm.at[vidx], vrows, sem_v)
        cu.wait()
        cv.wait()
        _dot_scores(urows, vrows, scores)
        pltpu.sync_copy(scores, out_hbm.at[pl.ds(off, CHUNK)])
        return carry

    lax.fori_loop(0, N_CHUNKS, chunk_body, 0)


@jax.jit
def kernel(h, edge_index):
    ei = edge_index.astype(jnp.int32)
    u = ei[0]
    v = ei[1]

    mesh = plsc.VectorSubcoreMesh(
        core_axis_name="c", subcore_axis_name="s",
        num_cores=NUM_CORES, num_subcores=NUM_SUBCORES,
    )
    run = functools.partial(
        pl.kernel,
        out_type=jax.ShapeDtypeStruct((N_EDGES,), jnp.float32),
        mesh=mesh,
        compiler_params=pltpu.CompilerParams(needs_layout_passes=False),
        scratch_types=[
            pltpu.VMEM((CHUNK,), jnp.int32),
            pltpu.VMEM((CHUNK,), jnp.int32),
            pltpu.VMEM((CHUNK, D_FEAT), jnp.float32),
            pltpu.VMEM((CHUNK, D_FEAT), jnp.float32),
            pltpu.VMEM((CHUNK,), jnp.float32),
            pltpu.SemaphoreType.DMA,
            pltpu.SemaphoreType.DMA,
        ],
    )(_sc_body)
    return run(h, u, v)


# speedup vs baseline: 1.3428x; 1.2161x over previous
import functools

import jax
import jax.numpy as jnp
from jax import lax
from jax.experimental import pallas as pl
from jax.experimental.pallas import tpu as pltpu
from jax.experimental.pallas import tpu_sc as plsc

N_NODES = 10000
N_EDGES = 320000
D_FEAT = 128

NUM_CORES = 2
NUM_SUBCORES = 16
LANES = 16
NUM_WORKERS = NUM_CORES * NUM_SUBCORES

E_PER_W = N_EDGES // NUM_WORKERS
CHUNK = 80
N_CHUNKS = E_PER_W // CHUNK
GROUPS = CHUNK // LANES


def _sc_body(h_hbm, u_hbm, v_hbm, out_hbm,
             uidx, vidx, scores,
             urows0, urows1, vrows0, vrows1, sem0, sem1):
    wid = lax.axis_index("s") * NUM_CORES + lax.axis_index("c")
    pltpu.sync_copy(u_hbm.at[wid], uidx)
    pltpu.sync_copy(v_hbm.at[wid], vidx)

    ubufs = (urows0, urows1)
    vbufs = (vrows0, vrows1)
    sems = (sem0, sem1)

    def issue(i, b):
        pltpu.async_copy(h_hbm.at[uidx.at[i]], ubufs[b], sems[b])
        pltpu.async_copy(h_hbm.at[vidx.at[i]], vbufs[b], sems[b])

    def wait(b):
        pltpu.make_async_copy(h_hbm.at[pl.ds(0, CHUNK)], ubufs[b], sems[b]).wait()
        pltpu.make_async_copy(h_hbm.at[pl.ds(0, CHUNK)], vbufs[b], sems[b]).wait()

    def compute(b, i):
        ur, vr = ubufs[b], vbufs[b]
        for g in range(GROUPS):
            eidx = g * LANES + lax.broadcasted_iota(jnp.int32, (LANES,), 0)

            def dbody(_, carry):
                acc, dvec = carry
                du = plsc.load_gather(ur, [eidx, dvec])
                dv = plsc.load_gather(vr, [eidx, dvec])
                return acc + du * dv, dvec + 1

            acc, _ = lax.fori_loop(
                0, D_FEAT, dbody,
                (jnp.zeros((LANES,), jnp.float32),
                 jnp.zeros((LANES,), jnp.int32)),
                unroll=8,
            )
            scores[i, pl.ds(g * LANES, LANES)] = acc

    issue(0, 0)

    def loop_body(j, carry):
        i0 = 2 * j
        issue(i0 + 1, 1)
        wait(0)
        compute(0, i0)
        issue(i0 + 2, 0)
        wait(1)
        compute(1, i0 + 1)
        return carry

    lax.fori_loop(0, (N_CHUNKS - 1) // 2, loop_body, 0)
    wait(0)
    compute(0, N_CHUNKS - 1)

    pltpu.sync_copy(scores, out_hbm.at[wid])


@jax.jit
def kernel(h, edge_index):
    ei = edge_index.astype(jnp.int32)
    u3 = ei[0].reshape(NUM_WORKERS, N_CHUNKS, CHUNK)
    v3 = ei[1].reshape(NUM_WORKERS, N_CHUNKS, CHUNK)

    mesh = plsc.VectorSubcoreMesh(
        core_axis_name="c", subcore_axis_name="s",
        num_cores=NUM_CORES, num_subcores=NUM_SUBCORES,
    )
    run = functools.partial(
        pl.kernel,
        out_type=jax.ShapeDtypeStruct((NUM_WORKERS, N_CHUNKS, CHUNK),
                                      jnp.float32),
        mesh=mesh,
        compiler_params=pltpu.CompilerParams(needs_layout_passes=False),
        scratch_types=[
            pltpu.VMEM((N_CHUNKS, CHUNK), jnp.int32),
            pltpu.VMEM((N_CHUNKS, CHUNK), jnp.int32),
            pltpu.VMEM((N_CHUNKS, CHUNK), jnp.float32),
            pltpu.VMEM((CHUNK, D_FEAT), jnp.float32),
            pltpu.VMEM((CHUNK, D_FEAT), jnp.float32),
            pltpu.VMEM((CHUNK, D_FEAT), jnp.float32),
            pltpu.VMEM((CHUNK, D_FEAT), jnp.float32),
            pltpu.SemaphoreType.DMA,
            pltpu.SemaphoreType.DMA,
        ],
    )(_sc_body)
    out3 = run(h, u3, v3)
    return out3.reshape(N_EDGES)


# speedup vs baseline: 6.5907x; 4.9081x over previous
import functools

import jax
import jax.numpy as jnp
from jax import lax
from jax.experimental import pallas as pl
from jax.experimental.pallas import tpu as pltpu
from jax.experimental.pallas import tpu_sc as plsc

N_NODES = 10000
N_EDGES = 320000
D_FEAT = 128

NUM_CORES = 2
NUM_SUBCORES = 16
LANES = 16
NUM_WORKERS = NUM_CORES * NUM_SUBCORES

E_PER_W = N_EDGES // NUM_WORKERS
CHUNK = 80
N_CHUNKS = E_PER_W // CHUNK
GROUPS = CHUNK // LANES


def _sc_body(h_hbm, u_hbm, v_hbm, out_hbm,
             uidx, vidx, scores, psum,
             urows0, urows1, vrows0, vrows1, sem0, sem1):
    wid = lax.axis_index("s") * NUM_CORES + lax.axis_index("c")
    pltpu.sync_copy(u_hbm.at[wid], uidx)
    pltpu.sync_copy(v_hbm.at[wid], vidx)

    ubufs = (urows0, urows1)
    vbufs = (vrows0, vrows1)
    sems = (sem0, sem1)

    def issue(i, b):
        pltpu.async_copy(h_hbm.at[uidx.at[i]], ubufs[b], sems[b])
        pltpu.async_copy(h_hbm.at[vidx.at[i]], vbufs[b], sems[b])

    def wait(b):
        pltpu.make_async_copy(h_hbm.at[pl.ds(0, CHUNK)], ubufs[b], sems[b]).wait()
        pltpu.make_async_copy(h_hbm.at[pl.ds(0, CHUNK)], vbufs[b], sems[b]).wait()

    lane_iota = lax.broadcasted_iota(jnp.int32, (LANES,), 0)

    def compute(b, i):
        ur, vr = ubufs[b], vbufs[b]

        def group_body(g, carry):
            def edge_body(el, c2):
                e = g * LANES + el
                acc = ur[e, pl.ds(0, LANES)] * vr[e, pl.ds(0, LANES)]
                for k in range(1, D_FEAT // LANES):
                    acc = acc + (ur[e, pl.ds(k * LANES, LANES)]
                                 * vr[e, pl.ds(k * LANES, LANES)])
                psum[el, pl.ds(0, LANES)] = acc
                return c2

            lax.fori_loop(0, LANES, edge_body, 0, unroll=8)
            tot = jnp.zeros((LANES,), jnp.float32)
            for l in range(LANES):
                tot = tot + plsc.load_gather(
                    psum, [lane_iota, jnp.full((LANES,), l, jnp.int32)])
            scores[i, pl.ds(g * LANES, LANES)] = tot
            return carry

        lax.fori_loop(0, GROUPS, group_body, 0)

    issue(0, 0)

    def loop_body(j, carry):
        i0 = 2 * j
        issue(i0 + 1, 1)
        wait(0)
        compute(0, i0)
        issue(i0 + 2, 0)
        wait(1)
        compute(1, i0 + 1)
        return carry

    lax.fori_loop(0, (N_CHUNKS - 1) // 2, loop_body, 0)
    wait(0)
    compute(0, N_CHUNKS - 1)

    pltpu.sync_copy(scores, out_hbm.at[wid])


@jax.jit
def kernel(h, edge_index):
    ei = edge_index.astype(jnp.int32)
    u3 = ei[0].reshape(NUM_WORKERS, N_CHUNKS, CHUNK)
    v3 = ei[1].reshape(NUM_WORKERS, N_CHUNKS, CHUNK)

    mesh = plsc.VectorSubcoreMesh(
        core_axis_name="c", subcore_axis_name="s",
        num_cores=NUM_CORES, num_subcores=NUM_SUBCORES,
    )
    run = functools.partial(
        pl.kernel,
        out_type=jax.ShapeDtypeStruct((NUM_WORKERS, N_CHUNKS, CHUNK),
                                      jnp.float32),
        mesh=mesh,
        compiler_params=pltpu.CompilerParams(needs_layout_passes=False),
        scratch_types=[
            pltpu.VMEM((N_CHUNKS, CHUNK), jnp.int32),
            pltpu.VMEM((N_CHUNKS, CHUNK), jnp.int32),
            pltpu.VMEM((N_CHUNKS, CHUNK), jnp.float32),
            pltpu.VMEM((LANES, 17), jnp.float32),
            pltpu.VMEM((CHUNK, D_FEAT), jnp.float32),
            pltpu.VMEM((CHUNK, D_FEAT), jnp.float32),
            pltpu.VMEM((CHUNK, D_FEAT), jnp.float32),
            pltpu.VMEM((CHUNK, D_FEAT), jnp.float32),
            pltpu.SemaphoreType.DMA,
            pltpu.SemaphoreType.DMA,
        ],
    )(_sc_body)
    out3 = run(h, u3, v3)
    return out3.reshape(N_EDGES)
